# baseline (speedup 1.0000x reference)
import jax
import jax.numpy as jnp
from jax.experimental import pallas as pl
from jax.experimental.pallas import tpu as pltpu

B = 2048
HID = 1024
E = 8
GD = 16
COEF = 0.01
TB = 8
BT = B // TB
S = E + 2


def _moe_body(x_ref, Wg1_ref, bg1_ref, Wg2_ref, bg2_ref, Wsh_ref, bsh_ref,
              shw_ref, rtw_ref, We_ref, be_ref, noise_ref,
              out_ref, loss_ref, m_ref, acc_ref, gsum_ref):
    s = pl.program_id(0)
    tb = pl.program_id(1)
    x = x_ref[...]

    @pl.when(s == 0)
    def _gating():
        h = jnp.maximum(
            jnp.dot(x, Wg1_ref[...].T, preferred_element_type=jnp.float32)
            + bg1_ref[...], 0.0)
        logits = (jnp.dot(h, Wg2_ref[...].T,
                          preferred_element_type=jnp.float32) + bg2_ref[...])
        gw = jax.nn.softmax(logits, axis=-1) + noise_ref[...]
        e_iota = jax.lax.broadcasted_iota(jnp.int32, (BT, E), 1)
        i1 = jnp.argmax(gw, axis=1)
        m1 = jnp.max(gw, axis=1)
        masked = jnp.where(e_iota == i1[:, None], -jnp.inf, gw)
        i2 = jnp.argmax(masked, axis=1)
        m2 = jnp.max(masked, axis=1)
        denom = m1 + m2
        w1 = (m1 / denom)[:, None]
        w2 = (m2 / denom)[:, None]
        m = (jnp.where(e_iota == i1[:, None], w1, 0.0)
             + jnp.where(e_iota == i2[:, None], w2, 0.0))
        m_ref[pl.ds(tb * BT, BT), :] = m

        @pl.when(tb == 0)
        def _():
            gsum_ref[...] = jnp.zeros_like(gsum_ref)
        gsum_ref[...] += jnp.sum(gw, axis=0, keepdims=True)

    @pl.when((s >= 1) & (s <= E))
    def _expert():
        e = s - 1
        W = We_ref[0]
        z = (jnp.dot(x.astype(jnp.bfloat16), W.T.astype(jnp.bfloat16),
                     preferred_element_type=jnp.float32)
             + be_ref[0])
        y = jax.nn.sigmoid(z)
        m_blk = m_ref[pl.ds(tb * BT, BT), :]
        sel = (jax.lax.broadcasted_iota(jnp.int32, (BT, E), 1) == e)
        mcol = jnp.sum(jnp.where(sel, m_blk, 0.0), axis=1, keepdims=True)
        contrib = mcol * y

        @pl.when(s == 1)
        def _():
            acc_ref[pl.ds(tb * BT, BT), :] = x + contrib

        @pl.when(s > 1)
        def _():
            acc_ref[pl.ds(tb * BT, BT), :] += contrib

    @pl.when(s == S - 1)
    def _final():
        sh = jax.nn.sigmoid(
            jnp.dot(x.astype(jnp.bfloat16), Wsh_ref[...].T.astype(jnp.bfloat16),
                    preferred_element_type=jnp.float32)
            + bsh_ref[...])
        out_ref[...] = (x + shw_ref[0, 0] * sh
                        + rtw_ref[0, 0] * acc_ref[pl.ds(tb * BT, BT), :])

        @pl.when(tb == TB - 1)
        def _():
            p = gsum_ref[...] * (1.0 / B)
            loss_ref[...] = jnp.reshape(
                jnp.mean((1.0 / E - p) ** 2) * COEF, (1, 1))


def kernel(x, W_g1, b_g1, W_g2, b_g2, W_sh, b_sh, sh_w, rt_w, W_e, b_e, noise):
    const2 = lambda s, tb: (0, 0)
    out, loss = pl.pallas_call(
        _moe_body,
        grid=(S, TB),
        in_specs=[
            pl.BlockSpec((BT, HID), lambda s, tb: (tb, 0)),
            pl.BlockSpec((GD, HID), const2),
            pl.BlockSpec((1, GD), const2),
            pl.BlockSpec((E, GD), const2),
            pl.BlockSpec((1, E), const2),
            pl.BlockSpec((HID, HID), const2),
            pl.BlockSpec((1, HID), const2),
            pl.BlockSpec((1, 1), const2),
            pl.BlockSpec((1, 1), const2),
            pl.BlockSpec((1, HID, HID),
                         lambda s, tb: (jnp.clip(s - 1, 0, E - 1), 0, 0)),
            pl.BlockSpec((1, 1, HID),
                         lambda s, tb: (jnp.clip(s - 1, 0, E - 1), 0, 0)),
            pl.BlockSpec((BT, E), lambda s, tb: (tb, 0)),
        ],
        out_specs=[
            pl.BlockSpec((BT, HID), lambda s, tb: (tb, 0)),
            pl.BlockSpec((1, 1), const2),
        ],
        out_shape=[
            jax.ShapeDtypeStruct((B, HID), jnp.float32),
            jax.ShapeDtypeStruct((1, 1), jnp.float32),
        ],
        scratch_shapes=[
            pltpu.VMEM((B, E), jnp.float32),
            pltpu.VMEM((B, HID), jnp.float32),
            pltpu.VMEM((1, E), jnp.float32),
        ],
    )(x, W_g1, b_g1.reshape(1, GD), W_g2, b_g2.reshape(1, E),
      W_sh, b_sh.reshape(1, HID), sh_w.reshape(1, 1), rt_w.reshape(1, 1),
      W_e, b_e.reshape(E, 1, HID), noise)
    return out, loss[0, 0]


# speedup vs baseline: 1.8482x; 1.8482x over previous
import jax
import jax.numpy as jnp
from jax.experimental import pallas as pl
from jax.experimental.pallas import tpu as pltpu

B = 2048
HID = 1024
E = 8
GD = 16
COEF = 0.01
S = E + 2


def _moe_body(x_ref, Wg1_ref, bg1_ref, Wg2_ref, bg2_ref, Wsh_ref, bsh_ref,
              shw_ref, rtw_ref, We_ref, be_ref, noise_ref,
              out_ref, loss_ref, m_ref, gsum_ref):
    s = pl.program_id(0)

    @pl.when(s == 0)
    def _gating():
        x = x_ref[...]
        h = jnp.maximum(
            jnp.dot(x, Wg1_ref[...].T, preferred_element_type=jnp.float32)
            + bg1_ref[...], 0.0)
        logits = (jnp.dot(h, Wg2_ref[...].T,
                          preferred_element_type=jnp.float32) + bg2_ref[...])
        gw = jax.nn.softmax(logits, axis=-1) + noise_ref[...]
        e_iota = jax.lax.broadcasted_iota(jnp.int32, (B, E), 1)
        i1 = jnp.argmax(gw, axis=1)
        m1 = jnp.max(gw, axis=1)
        masked = jnp.where(e_iota == i1[:, None], -jnp.inf, gw)
        i2 = jnp.argmax(masked, axis=1)
        m2 = jnp.max(masked, axis=1)
        denom = m1 + m2
        w1 = (m1 / denom)[:, None]
        w2 = (m2 / denom)[:, None]
        m_ref[...] = (jnp.where(e_iota == i1[:, None], w1, 0.0)
                      + jnp.where(e_iota == i2[:, None], w2, 0.0))
        gsum_ref[...] = jnp.sum(gw, axis=0, keepdims=True)

    @pl.when((s >= 1) & (s <= E))
    def _expert():
        e = s - 1
        x = x_ref[...]
        W = We_ref[0]
        z = jnp.dot(x, W.T, preferred_element_type=jnp.float32) + be_ref[0]
        y = jax.nn.sigmoid(z)
        sel = (jax.lax.broadcasted_iota(jnp.int32, (B, E), 1) == e)
        mcol = jnp.sum(jnp.where(sel, m_ref[...], 0.0), axis=1, keepdims=True)
        contrib = mcol * y

        @pl.when(s == 1)
        def _():
            out_ref[...] = contrib

        @pl.when(s > 1)
        def _():
            out_ref[...] += contrib

    @pl.when(s == S - 1)
    def _final():
        x = x_ref[...]
        sh = jax.nn.sigmoid(
            jnp.dot(x, Wsh_ref[...].T, preferred_element_type=jnp.float32)
            + bsh_ref[...])
        out_ref[...] = (x + shw_ref[0, 0] * sh
                        + rtw_ref[0, 0] * (out_ref[...] + x))
        p = gsum_ref[...] * (1.0 / B)
        loss_ref[...] = jnp.reshape(jnp.mean((1.0 / E - p) ** 2) * COEF,
                                    (1, 1))


def kernel(x, W_g1, b_g1, W_g2, b_g2, W_sh, b_sh, sh_w, rt_w, W_e, b_e, noise):
    const2 = lambda s: (0, 0)
    out, loss = pl.pallas_call(
        _moe_body,
        grid=(S,),
        in_specs=[
            pl.BlockSpec((B, HID), const2),
            pl.BlockSpec((GD, HID), const2),
            pl.BlockSpec((1, GD), const2),
            pl.BlockSpec((E, GD), const2),
            pl.BlockSpec((1, E), const2),
            pl.BlockSpec((HID, HID), const2),
            pl.BlockSpec((1, HID), const2),
            pl.BlockSpec((1, 1), const2),
            pl.BlockSpec((1, 1), const2),
            pl.BlockSpec((1, HID, HID),
                         lambda s: (jnp.clip(s - 1, 0, E - 1), 0, 0)),
            pl.BlockSpec((1, 1, HID),
                         lambda s: (jnp.clip(s - 1, 0, E - 1), 0, 0)),
            pl.BlockSpec((B, E), const2),
        ],
        out_specs=[
            pl.BlockSpec((B, HID), const2),
            pl.BlockSpec((1, 1), const2),
        ],
        out_shape=[
            jax.ShapeDtypeStruct((B, HID), jnp.float32),
            jax.ShapeDtypeStruct((1, 1), jnp.float32),
        ],
        scratch_shapes=[
            pltpu.VMEM((B, E), jnp.float32),
            pltpu.VMEM((1, E), jnp.float32),
        ],
    )(x, W_g1, b_g1.reshape(1, GD), W_g2, b_g2.reshape(1, E),
      W_sh, b_sh.reshape(1, HID), sh_w.reshape(1, 1), rt_w.reshape(1, 1),
      W_e, b_e.reshape(E, 1, HID), noise)
    return out, loss[0, 0]
